# Initial kernel scaffold; baseline (speedup 1.0000x reference)
import jax
import jax.numpy as jnp
from jax.experimental import pallas as pl


def kernel(x, edge_index, W1, b1, W2, b2, W3, b3):
    raise NotImplementedError("write your pallas kernel here")


# speedup vs baseline: 31.2239x; 31.2239x over previous
import functools

import jax
import jax.numpy as jnp
from jax import lax
from jax.experimental import pallas as pl
from jax.experimental.pallas import tpu as pltpu
from jax.experimental.pallas import tpu_sc as plsc

N_NODES = 10000
NPAD = 10240
D_FEAT = 128
NC = 2
NS = 16
CH = 128
ROWS_PT = NPAD // NS
WIDTH = 16
BLK = 1024


def _sc_mesh():
    return plsc.VectorSubcoreMesh(core_axis_name="c", subcore_axis_name="s")


@functools.cache
def _sc_scatter_rows(K, width):
    def body(table, src_hbm, dst_hbm, zeros_hbm, out, src_v, dst_v, buf, accum, sem):
        c = lax.axis_index("c")
        s = lax.axis_index("s")
        pltpu.sync_copy(zeros_hbm, accum.at[pl.ds(s * ROWS_PT, ROWS_PT)])
        pltpu.sync_copy(src_hbm.at[c, s], src_v)
        pltpu.sync_copy(dst_hbm.at[c, s], dst_v)
        plsc.subcore_barrier()

        @pl.loop(0, K)
        def _(j):
            pltpu.async_copy(table.at[src_v.at[j]], buf, sem).wait()
            pltpu.sync_copy(buf, accum.at[dst_v.at[j]], add=True)

        plsc.subcore_barrier()
        pltpu.sync_copy(
            accum.at[pl.ds(s * ROWS_PT, ROWS_PT)],
            out.at[c, pl.ds(s * ROWS_PT, ROWS_PT)],
        )

    return pl.kernel(
        body,
        out_type=jax.ShapeDtypeStruct((NC, NPAD, width), jnp.float32),
        mesh=_sc_mesh(),
        scratch_types=[
            pltpu.VMEM((K, CH), jnp.int32),
            pltpu.VMEM((K, CH), jnp.int32),
            pltpu.VMEM((CH, width), jnp.float32),
            pltpu.VMEM_SHARED((NPAD, width), jnp.float32),
            pltpu.SemaphoreType.DMA,
        ],
        compiler_params=pltpu.CompilerParams(use_tc_tiling_on_sc=False),
    )


@functools.cache
def _sc_degree(K, width):
    def body(ones_hbm, dst_hbm, zeros_hbm, out, dst_v, buf, accum, sem):
        c = lax.axis_index("c")
        s = lax.axis_index("s")
        pltpu.sync_copy(zeros_hbm, accum.at[pl.ds(s * ROWS_PT, ROWS_PT)])
        pltpu.sync_copy(ones_hbm, buf)
        pltpu.sync_copy(dst_hbm.at[c, s], dst_v)
        plsc.subcore_barrier()

        @pl.loop(0, K)
        def _(j):
            pltpu.sync_copy(buf, accum.at[dst_v.at[j]], add=True)

        plsc.subcore_barrier()
        pltpu.sync_copy(
            accum.at[pl.ds(s * ROWS_PT, ROWS_PT)],
            out.at[c, pl.ds(s * ROWS_PT, ROWS_PT)],
        )

    return pl.kernel(
        body,
        out_type=jax.ShapeDtypeStruct((NC, NPAD, width), jnp.float32),
        mesh=_sc_mesh(),
        scratch_types=[
            pltpu.VMEM((K, CH), jnp.int32),
            pltpu.VMEM((CH, width), jnp.float32),
            pltpu.VMEM_SHARED((NPAD, width), jnp.float32),
            pltpu.SemaphoreType.DMA,
        ],
        compiler_params=pltpu.CompilerParams(use_tc_tiling_on_sc=False),
    )


def _tc_xw(x, W1):
    def body(x_ref, w_ref, o_ref):
        o_ref[...] = jnp.dot(x_ref[...], w_ref[...], preferred_element_type=jnp.float32)

    return pl.pallas_call(
        body,
        grid=(NPAD // BLK,),
        in_specs=[
            pl.BlockSpec((BLK, D_FEAT), lambda i: (i, 0)),
            pl.BlockSpec((D_FEAT, 16), lambda i: (0, 0)),
        ],
        out_specs=pl.BlockSpec((BLK, 16), lambda i: (i, 0)),
        out_shape=jax.ShapeDtypeStruct((NPAD, 16), jnp.float32),
    )(x, W1)


def _tc_dinv_scale(deg_p, t1):
    def body(d_ref, t_ref, dinv_ref, g_ref):
        dp = d_ref[0] + d_ref[1] + 1.0
        dinv = lax.rsqrt(dp)
        dinv_ref[...] = dinv[:, 0:1]
        g_ref[...] = t_ref[...] * dinv

    return pl.pallas_call(
        body,
        grid=(NPAD // BLK,),
        in_specs=[
            pl.BlockSpec((NC, BLK, WIDTH), lambda i: (0, i, 0)),
            pl.BlockSpec((BLK, 16), lambda i: (i, 0)),
        ],
        out_specs=[
            pl.BlockSpec((BLK, 1), lambda i: (i, 0)),
            pl.BlockSpec((BLK, 16), lambda i: (i, 0)),
        ],
        out_shape=[
            jax.ShapeDtypeStruct((NPAD, 1), jnp.float32),
            jax.ShapeDtypeStruct((NPAD, 16), jnp.float32),
        ],
    )(deg_p, t1)


def _tc_layer1_finish(s1_p, g1, dinv, W2, b1):
    def body(p_ref, g_ref, di_ref, w_ref, b_ref, o_ref):
        s = p_ref[0] + p_ref[1] + g_ref[...]
        z = jnp.maximum(s * di_ref[...] + b_ref[...], 0.0)
        t2 = jnp.dot(z, w_ref[...], preferred_element_type=jnp.float32)
        g2 = t2 * di_ref[...]
        o_ref[...] = jnp.concatenate([g2, jnp.zeros_like(g2)], axis=1)

    return pl.pallas_call(
        body,
        grid=(NPAD // BLK,),
        in_specs=[
            pl.BlockSpec((NC, BLK, WIDTH), lambda i: (0, i, 0)),
            pl.BlockSpec((BLK, 16), lambda i: (i, 0)),
            pl.BlockSpec((BLK, 1), lambda i: (i, 0)),
            pl.BlockSpec((16, 8), lambda i: (0, 0)),
            pl.BlockSpec((1, 16), lambda i: (0, 0)),
        ],
        out_specs=pl.BlockSpec((BLK, 16), lambda i: (i, 0)),
        out_shape=jax.ShapeDtypeStruct((NPAD, 16), jnp.float32),
    )(s1_p, g1, dinv, W2, b1)


def _tc_layer2_head(s2_p, g2, dinv, W3, b2, b3):
    def body(p_ref, g_ref, di_ref, w_ref, b2_ref, b3_ref, o_ref):
        s = (p_ref[0] + p_ref[1] + g_ref[...])[:, 0:8]
        z = jnp.maximum(s * di_ref[...] + b2_ref[...], 0.0)
        o_ref[...] = (
            jnp.dot(z, w_ref[...], preferred_element_type=jnp.float32) + b3_ref[...]
        )

    return pl.pallas_call(
        body,
        grid=(NPAD // BLK,),
        in_specs=[
            pl.BlockSpec((NC, BLK, WIDTH), lambda i: (0, i, 0)),
            pl.BlockSpec((BLK, 16), lambda i: (i, 0)),
            pl.BlockSpec((BLK, 1), lambda i: (i, 0)),
            pl.BlockSpec((8, 1), lambda i: (0, 0)),
            pl.BlockSpec((1, 8), lambda i: (0, 0)),
            pl.BlockSpec((1, 1), lambda i: (0, 0)),
        ],
        out_specs=pl.BlockSpec((BLK, 1), lambda i: (i, 0)),
        out_shape=jax.ShapeDtypeStruct((NPAD, 1), jnp.float32),
    )(s2_p, g2, dinv, W3, b2, b3)


def kernel(x, edge_index, W1, b1, W2, b2, W3, b3):
    n_edges = edge_index.shape[1]
    K = -(-n_edges // (NC * NS * CH))
    epad = NC * NS * K * CH

    src = edge_index[0].astype(jnp.int32)
    dst = edge_index[1].astype(jnp.int32)
    fill = jnp.full((epad - n_edges,), N_NODES, jnp.int32)
    src4 = jnp.concatenate([src, fill]).reshape(NC, NS, K, CH)
    dst4 = jnp.concatenate([dst, fill]).reshape(NC, NS, K, CH)

    x_pad = jnp.pad(x, ((0, NPAD - N_NODES), (0, 0)))
    zeros = jnp.zeros((ROWS_PT, WIDTH), jnp.float32)
    ones = jnp.ones((CH, WIDTH), jnp.float32)

    deg_p = _sc_degree(K, WIDTH)(ones, dst4, zeros)
    t1 = _tc_xw(x_pad, W1)
    dinv, g1 = _tc_dinv_scale(deg_p, t1)

    s1_p = _sc_scatter_rows(K, WIDTH)(g1, src4, dst4, zeros)
    g2 = _tc_layer1_finish(s1_p, g1, dinv, W2, b1.reshape(1, 16))

    s2_p = _sc_scatter_rows(K, WIDTH)(g2, src4, dst4, zeros)
    out = _tc_layer2_head(s2_p, g2, dinv, W3, b2.reshape(1, 8), b3.reshape(1, 1))

    return out[:N_NODES]


# speedup vs baseline: 40.6746x; 1.3027x over previous
import functools

import jax
import jax.numpy as jnp
from jax import lax
from jax.experimental import pallas as pl
from jax.experimental.pallas import tpu as pltpu
from jax.experimental.pallas import tpu_sc as plsc

N_NODES = 10000
NPAD = 10240
D_FEAT = 128
NC = 2
NS = 16
CH = 128
ROWS_PT = NPAD // NS
BLK = 1024
NBUF = 4


def _sc_mesh():
    return plsc.VectorSubcoreMesh(core_axis_name="c", subcore_axis_name="s")


@functools.cache
def _sc_scatter_rows(K, width):
    def body(table, src_hbm, dst_hbm, zeros_hbm, out, src_v, dst_v, bufs, accum, sems):
        c = lax.axis_index("c")
        s = lax.axis_index("s")
        pltpu.sync_copy(zeros_hbm, accum.at[pl.ds(s * ROWS_PT, ROWS_PT)])
        pltpu.sync_copy(src_hbm.at[c, s], src_v)
        pltpu.sync_copy(dst_hbm.at[c, s], dst_v)
        plsc.subcore_barrier()

        for b in range(NBUF):
            pltpu.async_copy(table.at[src_v.at[b]], bufs[b], sems[b])

        @pl.loop(0, K, step=NBUF)
        def _(i):
            for b in range(NBUF):
                j = i + b
                pltpu.make_async_copy(table.at[src_v.at[j]], bufs[b], sems[b]).wait()
                pltpu.sync_copy(bufs[b], accum.at[dst_v.at[j]], add=True)

                @pl.when(j + NBUF < K)
                def _():
                    pltpu.async_copy(table.at[src_v.at[j + NBUF]], bufs[b], sems[b])

        plsc.subcore_barrier()
        pltpu.sync_copy(
            accum.at[pl.ds(s * ROWS_PT, ROWS_PT)],
            out.at[c, pl.ds(s * ROWS_PT, ROWS_PT)],
        )

    return pl.kernel(
        body,
        out_type=jax.ShapeDtypeStruct((NC, NPAD, width), jnp.float32),
        mesh=_sc_mesh(),
        scratch_types=[
            pltpu.VMEM((K, CH), jnp.int32),
            pltpu.VMEM((K, CH), jnp.int32),
            [pltpu.VMEM((CH, width), jnp.float32) for _ in range(NBUF)],
            pltpu.VMEM_SHARED((NPAD, width), jnp.float32),
            [pltpu.SemaphoreType.DMA for _ in range(NBUF)],
        ],
        compiler_params=pltpu.CompilerParams(use_tc_tiling_on_sc=False),
    )


@functools.cache
def _sc_degree(K, width):
    def body(ones_hbm, dst_hbm, zeros_hbm, out, dst_v, buf, accum, sem):
        c = lax.axis_index("c")
        s = lax.axis_index("s")
        pltpu.sync_copy(zeros_hbm, accum.at[pl.ds(s * ROWS_PT, ROWS_PT)])
        pltpu.sync_copy(ones_hbm, buf)
        pltpu.sync_copy(dst_hbm.at[c, s], dst_v)
        plsc.subcore_barrier()

        @pl.loop(0, K, step=8)
        def _(i):
            for b in range(8):
                pltpu.async_copy(buf, accum.at[dst_v.at[i + b]], sem, add=True)
            for b in range(8):
                pltpu.make_async_copy(buf, accum.at[dst_v.at[i]], sem).wait()

        plsc.subcore_barrier()
        pltpu.sync_copy(
            accum.at[pl.ds(s * ROWS_PT, ROWS_PT)],
            out.at[c, pl.ds(s * ROWS_PT, ROWS_PT)],
        )

    return pl.kernel(
        body,
        out_type=jax.ShapeDtypeStruct((NC, NPAD, width), jnp.float32),
        mesh=_sc_mesh(),
        scratch_types=[
            pltpu.VMEM((K, CH), jnp.int32),
            pltpu.VMEM((CH, width), jnp.float32),
            pltpu.VMEM_SHARED((NPAD, width), jnp.float32),
            pltpu.SemaphoreType.DMA,
        ],
        compiler_params=pltpu.CompilerParams(use_tc_tiling_on_sc=False),
    )


def _tc_xw(x, W1):
    def body(x_ref, w_ref, o_ref):
        o_ref[...] = jnp.dot(x_ref[...], w_ref[...], preferred_element_type=jnp.float32)

    return pl.pallas_call(
        body,
        grid=(NPAD // BLK,),
        in_specs=[
            pl.BlockSpec((BLK, D_FEAT), lambda i: (i, 0)),
            pl.BlockSpec((D_FEAT, 16), lambda i: (0, 0)),
        ],
        out_specs=pl.BlockSpec((BLK, 16), lambda i: (i, 0)),
        out_shape=jax.ShapeDtypeStruct((NPAD, 16), jnp.float32),
    )(x, W1)


def _tc_dinv_scale(deg_p, t1):
    def body(d_ref, t_ref, dinv_ref, g_ref):
        dp = d_ref[0] + d_ref[1] + 1.0
        dinv = lax.rsqrt(dp)
        dinv_ref[...] = dinv
        g_ref[...] = t_ref[...] * dinv

    return pl.pallas_call(
        body,
        grid=(NPAD // BLK,),
        in_specs=[
            pl.BlockSpec((NC, BLK, 1), lambda i: (0, i, 0)),
            pl.BlockSpec((BLK, 16), lambda i: (i, 0)),
        ],
        out_specs=[
            pl.BlockSpec((BLK, 1), lambda i: (i, 0)),
            pl.BlockSpec((BLK, 16), lambda i: (i, 0)),
        ],
        out_shape=[
            jax.ShapeDtypeStruct((NPAD, 1), jnp.float32),
            jax.ShapeDtypeStruct((NPAD, 16), jnp.float32),
        ],
    )(deg_p, t1)


def _tc_layer1_finish(s1_p, g1, dinv, W2, b1):
    def body(p_ref, g_ref, di_ref, w_ref, b_ref, o_ref):
        s = p_ref[0] + p_ref[1] + g_ref[...]
        z = jnp.maximum(s * di_ref[...] + b_ref[...], 0.0)
        t2 = jnp.dot(z, w_ref[...], preferred_element_type=jnp.float32)
        o_ref[...] = t2 * di_ref[...]

    return pl.pallas_call(
        body,
        grid=(NPAD // BLK,),
        in_specs=[
            pl.BlockSpec((NC, BLK, 16), lambda i: (0, i, 0)),
            pl.BlockSpec((BLK, 16), lambda i: (i, 0)),
            pl.BlockSpec((BLK, 1), lambda i: (i, 0)),
            pl.BlockSpec((16, 8), lambda i: (0, 0)),
            pl.BlockSpec((1, 16), lambda i: (0, 0)),
        ],
        out_specs=pl.BlockSpec((BLK, 8), lambda i: (i, 0)),
        out_shape=jax.ShapeDtypeStruct((NPAD, 8), jnp.float32),
    )(s1_p, g1, dinv, W2, b1)


def _tc_layer2_head(s2_p, g2, dinv, W3, b2, b3):
    def body(p_ref, g_ref, di_ref, w_ref, b2_ref, b3_ref, o_ref):
        s = p_ref[0] + p_ref[1] + g_ref[...]
        z = jnp.maximum(s * di_ref[...] + b2_ref[...], 0.0)
        o_ref[...] = (
            jnp.dot(z, w_ref[...], preferred_element_type=jnp.float32) + b3_ref[...]
        )

    return pl.pallas_call(
        body,
        grid=(NPAD // BLK,),
        in_specs=[
            pl.BlockSpec((NC, BLK, 8), lambda i: (0, i, 0)),
            pl.BlockSpec((BLK, 8), lambda i: (i, 0)),
            pl.BlockSpec((BLK, 1), lambda i: (i, 0)),
            pl.BlockSpec((8, 1), lambda i: (0, 0)),
            pl.BlockSpec((1, 8), lambda i: (0, 0)),
            pl.BlockSpec((1, 1), lambda i: (0, 0)),
        ],
        out_specs=pl.BlockSpec((BLK, 1), lambda i: (i, 0)),
        out_shape=jax.ShapeDtypeStruct((NPAD, 1), jnp.float32),
    )(s2_p, g2, dinv, W3, b2, b3)


def kernel(x, edge_index, W1, b1, W2, b2, W3, b3):
    n_edges = edge_index.shape[1]
    K = -(-n_edges // (NC * NS * CH))
    K = -(-K // 8) * 8
    epad = NC * NS * K * CH

    src = edge_index[0].astype(jnp.int32)
    dst = edge_index[1].astype(jnp.int32)
    fill = jnp.full((epad - n_edges,), N_NODES, jnp.int32)
    src4 = jnp.concatenate([src, fill]).reshape(NC, NS, K, CH)
    dst4 = jnp.concatenate([dst, fill]).reshape(NC, NS, K, CH)

    x_pad = jnp.pad(x, ((0, NPAD - N_NODES), (0, 0)))
    zeros16 = jnp.zeros((ROWS_PT, 16), jnp.float32)
    zeros8 = jnp.zeros((ROWS_PT, 8), jnp.float32)
    zeros1 = jnp.zeros((ROWS_PT, 1), jnp.float32)
    ones1 = jnp.ones((CH, 1), jnp.float32)

    deg_p = _sc_degree(K, 1)(ones1, dst4, zeros1)
    t1 = _tc_xw(x_pad, W1)
    dinv, g1 = _tc_dinv_scale(deg_p, t1)

    s1_p = _sc_scatter_rows(K, 16)(g1, src4, dst4, zeros16)
    g2 = _tc_layer1_finish(s1_p, g1, dinv, W2, b1.reshape(1, 16))

    s2_p = _sc_scatter_rows(K, 8)(g2, src4, dst4, zeros8)
    out = _tc_layer2_head(s2_p, g2, dinv, W3, b2.reshape(1, 8), b3.reshape(1, 1))

    return out[:N_NODES]
